# baseline (speedup 1.0000x reference)
import jax
import jax.numpy as jnp
from jax import lax
from jax.experimental import pallas as pl

B, NU, NE, D = 64, 2048, 16, 256
G = 2


def _fused_kernel(user_h_ref, ego_h_ref, user_mask_ref, ego_mask_ref,
                  neigh_ref, hgs_ref, adj_ref):
    row = lax.broadcasted_iota(jnp.int32, (NE, NE), 0)
    col = lax.broadcasted_iota(jnp.int32, (NE, NE), 1)
    for g in range(G):
        mu = (user_mask_ref[g] > 0).astype(jnp.float32)
        uh = user_h_ref[g]
        num_u = jnp.dot(mu, uh, preferred_element_type=jnp.float32)
        cnt_u = jnp.maximum(jnp.sum(mu, axis=1, keepdims=True), 1.0)

        me = (ego_mask_ref[g] > 0).astype(jnp.float32)
        eh = ego_h_ref[g]
        num_e = jnp.dot(me, eh, preferred_element_type=jnp.float32)
        cnt_e = jnp.maximum(jnp.sum(me, axis=1, keepdims=True), 1.0)

        hgs_ref[g * NE:(g + 1) * NE, :] = num_u / cnt_u + num_e / cnt_e

        nf = (neigh_ref[g] > 0).astype(jnp.float32)
        ov = jnp.dot(nf, nf.T, preferred_element_type=jnp.float32)
        adj_ref[g] = ((ov > 0.0) & (row != col)).astype(jnp.int32)


def kernel(user_h, ego_h, user_mask, ego_mask, neigh):
    hgs, adj_i = pl.pallas_call(
        _fused_kernel,
        grid=(B // G,),
        in_specs=[
            pl.BlockSpec((G, NU, D), lambda b: (b, 0, 0)),
            pl.BlockSpec((G, NE, D), lambda b: (b, 0, 0)),
            pl.BlockSpec((G, NE, NU), lambda b: (b, 0, 0)),
            pl.BlockSpec((G, NE, NE), lambda b: (b, 0, 0)),
            pl.BlockSpec((G, NE, NU), lambda b: (b, 0, 0)),
        ],
        out_specs=[
            pl.BlockSpec((G * NE, D), lambda b: (b, 0)),
            pl.BlockSpec((G, NE, NE), lambda b: (b, 0, 0)),
        ],
        out_shape=[
            jax.ShapeDtypeStruct((B * NE, D), jnp.float32),
            jax.ShapeDtypeStruct((B, NE, NE), jnp.int32),
        ],
    )(user_h, ego_h, user_mask, ego_mask, neigh)
    return hgs, adj_i.astype(bool)


# speedup vs baseline: 1.0662x; 1.0662x over previous
import jax
import jax.numpy as jnp
from jax import lax
from jax.experimental import pallas as pl

B, NU, NE, D = 64, 2048, 16, 256
G = 4


def _fused_kernel(user_h_ref, ego_h_ref, user_mask_ref, ego_mask_ref,
                  neigh_ref, hgs_ref, adj_ref):
    row = lax.broadcasted_iota(jnp.int32, (NE, NE), 0)
    col = lax.broadcasted_iota(jnp.int32, (NE, NE), 1)
    for g in range(G):
        mu = (user_mask_ref[g] > 0).astype(jnp.float32)
        uh = user_h_ref[g]
        num_u = jnp.dot(mu, uh, preferred_element_type=jnp.float32)
        cnt_u = jnp.maximum(jnp.sum(mu, axis=1, keepdims=True), 1.0)

        me = (ego_mask_ref[g] > 0).astype(jnp.float32)
        eh = ego_h_ref[g]
        num_e = jnp.dot(me, eh, preferred_element_type=jnp.float32)
        cnt_e = jnp.maximum(jnp.sum(me, axis=1, keepdims=True), 1.0)

        hgs_ref[g * NE:(g + 1) * NE, :] = num_u / cnt_u + num_e / cnt_e

        nf = (neigh_ref[g] > 0).astype(jnp.float32)
        ov = jnp.dot(nf, nf.T, preferred_element_type=jnp.float32)
        adj_ref[g] = ((ov > 0.0) & (row != col)).astype(jnp.int32)


def kernel(user_h, ego_h, user_mask, ego_mask, neigh):
    hgs, adj_i = pl.pallas_call(
        _fused_kernel,
        grid=(B // G,),
        in_specs=[
            pl.BlockSpec((G, NU, D), lambda b: (b, 0, 0)),
            pl.BlockSpec((G, NE, D), lambda b: (b, 0, 0)),
            pl.BlockSpec((G, NE, NU), lambda b: (b, 0, 0)),
            pl.BlockSpec((G, NE, NE), lambda b: (b, 0, 0)),
            pl.BlockSpec((G, NE, NU), lambda b: (b, 0, 0)),
        ],
        out_specs=[
            pl.BlockSpec((G * NE, D), lambda b: (b, 0)),
            pl.BlockSpec((G, NE, NE), lambda b: (b, 0, 0)),
        ],
        out_shape=[
            jax.ShapeDtypeStruct((B * NE, D), jnp.float32),
            jax.ShapeDtypeStruct((B, NE, NE), jnp.int32),
        ],
    )(user_h, ego_h, user_mask, ego_mask, neigh)
    return hgs, adj_i.astype(bool)


# speedup vs baseline: 1.0752x; 1.0085x over previous
import jax
import jax.numpy as jnp
from jax import lax
from jax.experimental import pallas as pl

B, NU, NE, D = 64, 2048, 16, 256
G = 4


def _fused_kernel(user_h_ref, ego_h_ref, user_mask_ref, ego_mask_ref,
                  neigh_ref, hgs_ref, adj_ref):
    row = lax.broadcasted_iota(jnp.int32, (NE, NE), 0)
    col = lax.broadcasted_iota(jnp.int32, (NE, NE), 1)
    for g in range(G):
        mu = (user_mask_ref[g] > 0).astype(jnp.bfloat16)
        uh = user_h_ref[g].astype(jnp.bfloat16)
        num_u = jnp.dot(mu, uh, preferred_element_type=jnp.float32)
        cnt_u = jnp.maximum(
            jnp.sum(mu.astype(jnp.float32), axis=1, keepdims=True), 1.0)

        me = (ego_mask_ref[g] > 0).astype(jnp.float32)
        eh = ego_h_ref[g]
        num_e = jnp.dot(me, eh, preferred_element_type=jnp.float32)
        cnt_e = jnp.maximum(jnp.sum(me, axis=1, keepdims=True), 1.0)

        hgs_ref[g * NE:(g + 1) * NE, :] = num_u / cnt_u + num_e / cnt_e

        nf = (neigh_ref[g] > 0).astype(jnp.bfloat16)
        ov = jnp.dot(nf, nf.T, preferred_element_type=jnp.float32)
        adj_ref[g] = ((ov > 0.0) & (row != col)).astype(jnp.int32)


def kernel(user_h, ego_h, user_mask, ego_mask, neigh):
    hgs, adj_i = pl.pallas_call(
        _fused_kernel,
        grid=(B // G,),
        in_specs=[
            pl.BlockSpec((G, NU, D), lambda b: (b, 0, 0)),
            pl.BlockSpec((G, NE, D), lambda b: (b, 0, 0)),
            pl.BlockSpec((G, NE, NU), lambda b: (b, 0, 0)),
            pl.BlockSpec((G, NE, NE), lambda b: (b, 0, 0)),
            pl.BlockSpec((G, NE, NU), lambda b: (b, 0, 0)),
        ],
        out_specs=[
            pl.BlockSpec((G * NE, D), lambda b: (b, 0)),
            pl.BlockSpec((G, NE, NE), lambda b: (b, 0, 0)),
        ],
        out_shape=[
            jax.ShapeDtypeStruct((B * NE, D), jnp.float32),
            jax.ShapeDtypeStruct((B, NE, NE), jnp.int32),
        ],
    )(user_h, ego_h, user_mask, ego_mask, neigh)
    return hgs, adj_i.astype(bool)
